# Initial kernel scaffold; baseline (speedup 1.0000x reference)
import jax
import jax.numpy as jnp
from jax.experimental import pallas as pl


def kernel(x, u1_w, u1_b, u1_g, u1_bt, u2_w, u2_b, u2_g, u2_bt, u3_w, u3_b, u3_g, u3_bt, u4_w, u4_b, u4_g, u4_bt, u5_w, u5_b, u5_g, u5_bt, u6_w, u6_b, u6_g, u6_bt, fc1_w, fc1_b, fc2_w, fc2_b):
    raise NotImplementedError("write your pallas kernel here")


# speedup vs baseline: 13.2114x; 13.2114x over previous
import functools

import jax
import jax.numpy as jnp
from jax import lax
from jax.experimental import pallas as pl
from jax.experimental.pallas import tpu as pltpu

_EPS = 1e-5


def _pack(x_nchw, r, p):
    n, c, h, w = x_nchw.shape
    hp, wp = h + 2, w + 2
    xp = jnp.pad(x_nchw, ((0, 0), (0, 0), (1, 1), (1, 1)))
    xp = xp.reshape(n // p, p, c, hp, wp)
    xp = jnp.moveaxis(xp, 1, 3)
    xp = xp.reshape(n // p, c, hp, p * wp)
    xp = jnp.pad(xp, ((0, 0), (0, 0), (0, 0), (0, r - p * wp)))
    return xp.reshape(n // p, c, hp * r)


def _unpack_pool(y, hp, wp, r, p, pool):
    nb, co, _ = y.shape
    h, w = hp - 2, wp - 2
    t = y.reshape(nb, co, hp, r)
    t = t[:, :, 1:hp - 1, :p * wp].reshape(nb, co, h, p, wp)[..., 1:wp - 1]
    t = t.reshape(nb, co, h // pool, pool, p, w).max(axis=3)
    t = t.reshape(nb, co, h // pool, p, w // pool, pool).max(axis=5)
    t = jnp.moveaxis(t, 3, 1)
    return t.reshape(nb * p, co, h // pool, w // pool)


def _prep_w(w):
    co = w.shape[0]
    return jnp.transpose(w, (0, 2, 3, 1)).reshape(co, -1)


def _bn_coeffs(sums, gamma, beta, m):
    s = jnp.sum(sums, axis=0)
    mean = s[:, 0] / m
    var = jnp.maximum(s[:, 1] / m - mean * mean, 0.0)
    scale = gamma * lax.rsqrt(var + _EPS)
    shift = beta - mean * scale
    return scale.reshape(-1, 1), shift.reshape(-1, 1)


def _real_mask(l, r, hp, wp, p):
    lane = lax.broadcasted_iota(jnp.int32, (1, l), 1)
    row = lax.shift_right_logical(lane, r.bit_length() - 1)
    col = jnp.bitwise_and(lane, r - 1)
    ok_h = (row >= 1) & (row <= hp - 2)
    ok_w = (col >= 1) & (col <= wp - 2)
    for q in range(1, p):
        ok_w |= (col >= q * wp + 1) & (col <= q * wp + wp - 2)
    return (ok_h & ok_w).astype(jnp.float32)


def _fill_taps(p_ref, x2, c, r):
    l = x2.shape[1]
    bases = (pltpu.roll(x2, 1, 1), x2, pltpu.roll(x2, l - 1, 1))
    for i in range(3):
        for j in range(3):
            tap = pltpu.roll(bases[j], (-(i - 1) * r) % l, 1)
            t = i * 3 + j
            p_ref[t * c:(t + 1) * c, :] = tap


def _conv_y(p_ref, w_ref, x2, c, r):
    _fill_taps(p_ref, x2, c, r)
    return jnp.dot(w_ref[...], p_ref[...], preferred_element_type=jnp.float32)


def _stats_body(x_ref, w_ref, sums_ref, p_ref, *, c, r, hp, wp, p):
    x2 = x_ref[0]
    y = _conv_y(p_ref, w_ref, x2, c, r)
    ym = y * _real_mask(y.shape[1], r, hp, wp, p)
    sums_ref[0] = jnp.concatenate(
        [jnp.sum(ym, axis=1, keepdims=True),
         jnp.sum(ym * ym, axis=1, keepdims=True)], axis=1)


def _apply_body(x_ref, w_ref, sc_ref, sh_ref, o_ref, p_ref, *, c, r, hp, wp, p):
    x2 = x_ref[0]
    y = _conv_y(p_ref, w_ref, x2, c, r)
    mask = _real_mask(y.shape[1], r, hp, wp, p)
    o_ref[0] = jnp.maximum(y * sc_ref[...] + sh_ref[...], 0.0) * mask


def _apply_stats_body(x_ref, w1_ref, sc_ref, sh_ref, w2_ref, o_ref, sums_ref,
                      p1_ref, p2_ref, *, c1, c2, r, hp, wp, p):
    x2 = x_ref[0]
    y1 = _conv_y(p1_ref, w1_ref, x2, c1, r)
    mask = _real_mask(y1.shape[1], r, hp, wp, p)
    a1 = jnp.maximum(y1 * sc_ref[...] + sh_ref[...], 0.0) * mask
    o_ref[0] = a1
    y2 = _conv_y(p2_ref, w2_ref, a1, c2, r)
    ym = y2 * mask
    sums_ref[0] = jnp.concatenate(
        [jnp.sum(ym, axis=1, keepdims=True),
         jnp.sum(ym * ym, axis=1, keepdims=True)], axis=1)


_PARALLEL = pltpu.CompilerParams(dimension_semantics=("parallel",))


def _unit_stats(x, wk, *, co, r, hp, wp, p):
    nb, c, l = x.shape
    return pl.pallas_call(
        functools.partial(_stats_body, c=c, r=r, hp=hp, wp=wp, p=p),
        grid=(nb,),
        in_specs=[pl.BlockSpec((1, c, l), lambda g: (g, 0, 0)),
                  pl.BlockSpec((co, 9 * c), lambda g: (0, 0))],
        out_specs=pl.BlockSpec((1, co, 2), lambda g: (g, 0, 0)),
        out_shape=jax.ShapeDtypeStruct((nb, co, 2), jnp.float32),
        scratch_shapes=[pltpu.VMEM((9 * c, l), jnp.float32)],
        compiler_params=_PARALLEL,
    )(x, wk)


def _unit_apply(x, wk, scale, shift, *, co, r, hp, wp, p):
    nb, c, l = x.shape
    return pl.pallas_call(
        functools.partial(_apply_body, c=c, r=r, hp=hp, wp=wp, p=p),
        grid=(nb,),
        in_specs=[pl.BlockSpec((1, c, l), lambda g: (g, 0, 0)),
                  pl.BlockSpec((co, 9 * c), lambda g: (0, 0)),
                  pl.BlockSpec((co, 1), lambda g: (0, 0)),
                  pl.BlockSpec((co, 1), lambda g: (0, 0))],
        out_specs=pl.BlockSpec((1, co, l), lambda g: (g, 0, 0)),
        out_shape=jax.ShapeDtypeStruct((nb, co, l), jnp.float32),
        scratch_shapes=[pltpu.VMEM((9 * c, l), jnp.float32)],
        compiler_params=_PARALLEL,
    )(x, wk, scale, shift)


def _unit_apply_stats(x, w1k, scale, shift, w2k, *, c1o, c2o, r, hp, wp, p):
    nb, c1, l = x.shape
    return pl.pallas_call(
        functools.partial(_apply_stats_body, c1=c1, c2=c1o, r=r, hp=hp,
                          wp=wp, p=p),
        grid=(nb,),
        in_specs=[pl.BlockSpec((1, c1, l), lambda g: (g, 0, 0)),
                  pl.BlockSpec((c1o, 9 * c1), lambda g: (0, 0)),
                  pl.BlockSpec((c1o, 1), lambda g: (0, 0)),
                  pl.BlockSpec((c1o, 1), lambda g: (0, 0)),
                  pl.BlockSpec((c2o, 9 * c1o), lambda g: (0, 0))],
        out_specs=(pl.BlockSpec((1, c1o, l), lambda g: (g, 0, 0)),
                   pl.BlockSpec((1, c2o, 2), lambda g: (g, 0, 0))),
        out_shape=(jax.ShapeDtypeStruct((nb, c1o, l), jnp.float32),
                   jax.ShapeDtypeStruct((nb, c2o, 2), jnp.float32)),
        scratch_shapes=[pltpu.VMEM((9 * c1, l), jnp.float32),
                        pltpu.VMEM((9 * c1o, l), jnp.float32)],
        compiler_params=_PARALLEL,
    )(x, w1k, scale, shift, w2k)


def _unit6_body(p_ref, w_ref, g_ref, b_ref, o_ref):
    y = jnp.dot(w_ref[...], p_ref[...], preferred_element_type=jnp.float32)
    m = y.shape[1]
    mean = jnp.sum(y, axis=1, keepdims=True) / m
    var = jnp.maximum(jnp.sum(y * y, axis=1, keepdims=True) / m - mean * mean,
                      0.0)
    scale = g_ref[...] * lax.rsqrt(var + _EPS)
    shift = b_ref[...] - mean * scale
    o_ref[...] = jnp.maximum(y * scale + shift, 0.0)


def _fc_body(x_ref, w_ref, b_ref, o_ref):
    o_ref[...] = jnp.dot(x_ref[...], w_ref[...],
                         preferred_element_type=jnp.float32) + b_ref[...]


def kernel(x, u1_w, u1_b, u1_g, u1_bt, u2_w, u2_b, u2_g, u2_bt,
           u3_w, u3_b, u3_g, u3_bt, u4_w, u4_b, u4_g, u4_bt,
           u5_w, u5_b, u5_g, u5_bt, u6_w, u6_b, u6_g, u6_bt,
           fc1_w, fc1_b, fc2_w, fc2_b):
    del u1_b, u2_b, u3_b, u4_b, u5_b, u6_b
    n = x.shape[0]
    w1k, w2k, w3k = _prep_w(u1_w), _prep_w(u2_w), _prep_w(u3_w)
    w4k, w5k = _prep_w(u4_w), _prep_w(u5_w)

    g1 = dict(r=256, hp=226, wp=226, p=1)
    m12 = float(n * 224 * 224)
    x1 = _pack(x, 256, 1)
    s1 = _unit_stats(x1, w1k, co=12, **g1)
    sc1, sh1 = _bn_coeffs(s1, u1_g, u1_bt, m12)
    a1, s2 = _unit_apply_stats(x1, w1k, sc1, sh1, w2k, c1o=12, c2o=24, **g1)
    sc2, sh2 = _bn_coeffs(s2, u2_g, u2_bt, m12)
    a2 = _unit_apply(a1, w2k, sc2, sh2, co=24, **g1)

    g3 = dict(r=128, hp=114, wp=114, p=1)
    x3 = _pack(_unpack_pool(a2, 226, 226, 256, 1, 2), 128, 1)
    s3 = _unit_stats(x3, w3k, co=36, **g3)
    sc3, sh3 = _bn_coeffs(s3, u3_g, u3_bt, float(n * 112 * 112))
    a3 = _unit_apply(x3, w3k, sc3, sh3, co=36, **g3)

    g4 = dict(r=128, hp=58, wp=58, p=2)
    x4 = _pack(_unpack_pool(a3, 114, 114, 128, 1, 2), 128, 2)
    s4 = _unit_stats(x4, w4k, co=48, **g4)
    sc4, sh4 = _bn_coeffs(s4, u4_g, u4_bt, float(n * 56 * 56))
    a4 = _unit_apply(x4, w4k, sc4, sh4, co=48, **g4)

    g5 = dict(r=128, hp=30, wp=30, p=4)
    x5 = _pack(_unpack_pool(a4, 58, 58, 128, 2, 2), 128, 4)
    s5 = _unit_stats(x5, w5k, co=48, **g5)
    sc5, sh5 = _bn_coeffs(s5, u5_g, u5_bt, float(n * 28 * 28))
    a5 = _unit_apply(x5, w5k, sc5, sh5, co=48, **g5)

    x6 = _unpack_pool(a5, 30, 30, 128, 4, 4)
    taps = jnp.stack([x6[:, :, i:i + 3, j:j + 3]
                      for i in range(5) for j in range(5)], axis=0)
    p6 = jnp.transpose(taps, (0, 2, 1, 3, 4)).reshape(25 * 48, n * 9)
    w6k = jnp.transpose(u6_w, (0, 2, 3, 1)).reshape(96, 25 * 48)
    a6 = pl.pallas_call(
        _unit6_body,
        out_shape=jax.ShapeDtypeStruct((96, n * 9), jnp.float32),
    )(p6, w6k, u6_g.reshape(96, 1), u6_bt.reshape(96, 1))

    feat = jnp.transpose(a6.reshape(96, n, 9), (1, 0, 2)).reshape(n, 864)
    wf = jnp.concatenate([fc1_w, fc2_w], axis=0).T
    bf = jnp.concatenate([fc1_b, fc2_b]).reshape(1, 10)
    out = pl.pallas_call(
        _fc_body,
        out_shape=jax.ShapeDtypeStruct((n, 10), jnp.float32),
    )(feat, wf, bf)
    return out[:, :5], out[:, 5:]


# speedup vs baseline: 23.5293x; 1.7810x over previous
import functools

import jax
import jax.numpy as jnp
from jax import lax
from jax.experimental import pallas as pl
from jax.experimental.pallas import tpu as pltpu

_EPS = 1e-5


def _pack(x_nchw, r, p):
    n, c, h, w = x_nchw.shape
    hp, wp = h + 2, w + 2
    xp = jnp.pad(x_nchw, ((0, 0), (0, 0), (1, 1), (1, 1)))
    xp = xp.reshape(n // p, p, c, hp, wp)
    xp = jnp.moveaxis(xp, 1, 3)
    xp = xp.reshape(n // p, c, hp, p * wp)
    xp = jnp.pad(xp, ((0, 0), (0, 0), (0, 0), (0, r - p * wp)))
    return xp.reshape(n // p, c, hp * r)


def _unpack_pool(y, hp, wp, r, p, pool):
    nb, co, _ = y.shape
    h, w = hp - 2, wp - 2
    t = y.reshape(nb, co, hp, r)
    t = t[:, :, 1:hp - 1, :p * wp].reshape(nb, co, h, p, wp)[..., 1:wp - 1]
    t = t.reshape(nb, co, h // pool, pool, p, w).max(axis=3)
    t = t.reshape(nb, co, h // pool, p, w // pool, pool).max(axis=5)
    t = jnp.moveaxis(t, 3, 1)
    return t.reshape(nb * p, co, h // pool, w // pool)


def _prep_w(w):
    co = w.shape[0]
    return jnp.transpose(w, (0, 2, 3, 1)).reshape(co, -1)


def _bn_coeffs(sums, gamma, beta, m):
    s = jnp.sum(sums, axis=0)
    mean = s[:, 0] / m
    var = jnp.maximum(s[:, 1] / m - mean * mean, 0.0)
    scale = gamma * lax.rsqrt(var + _EPS)
    shift = beta - mean * scale
    return scale.reshape(-1, 1), shift.reshape(-1, 1)


def _real_mask(l, r, hp, wp, p):
    lane = lax.broadcasted_iota(jnp.int32, (1, l), 1)
    row = lax.shift_right_logical(lane, r.bit_length() - 1)
    col = jnp.bitwise_and(lane, r - 1)
    ok_h = (row >= 1) & (row <= hp - 2)
    ok_w = (col >= 1) & (col <= wp - 2)
    for q in range(1, p):
        ok_w |= (col >= q * wp + 1) & (col <= q * wp + wp - 2)
    return (ok_h & ok_w).astype(jnp.float32)


def _fill_taps(p_ref, x2, c, r):
    l = x2.shape[1]
    bases = (pltpu.roll(x2, 1, 1), x2, pltpu.roll(x2, l - 1, 1))
    for i in range(3):
        for j in range(3):
            tap = pltpu.roll(bases[j], (-(i - 1) * r) % l, 1)
            t = i * 3 + j
            p_ref[t * c:(t + 1) * c, :] = tap


def _conv_y(p_ref, w_ref, x2, c, r):
    _fill_taps(p_ref, x2, c, r)
    return jnp.dot(w_ref[...], p_ref[...], preferred_element_type=jnp.float32)


def _stats_body(x_ref, w_ref, sums_ref, p_ref, *, c, r, hp, wp, p):
    x2 = x_ref[0]
    y = _conv_y(p_ref, w_ref, x2, c, r)
    ym = y * _real_mask(y.shape[1], r, hp, wp, p)
    sums_ref[0] = jnp.concatenate(
        [jnp.sum(ym, axis=1, keepdims=True),
         jnp.sum(ym * ym, axis=1, keepdims=True)], axis=1)


def _apply_body(x_ref, w_ref, sc_ref, sh_ref, o_ref, p_ref, *, c, r, hp, wp, p):
    x2 = x_ref[0]
    y = _conv_y(p_ref, w_ref, x2, c, r)
    mask = _real_mask(y.shape[1], r, hp, wp, p)
    a = jnp.maximum(y * sc_ref[...] + sh_ref[...], 0.0) * mask
    o_ref[0] = a[:, r:(hp - 1) * r]


def _apply_stats_body(x_ref, w1_ref, sc_ref, sh_ref, w2_ref, o_ref, sums_ref,
                      p1_ref, p2_ref, *, c1, c2, r, hp, wp, p):
    x2 = x_ref[0]
    y1 = _conv_y(p1_ref, w1_ref, x2, c1, r)
    mask = _real_mask(y1.shape[1], r, hp, wp, p)
    a1 = jnp.maximum(y1 * sc_ref[...] + sh_ref[...], 0.0) * mask
    o_ref[0] = a1
    y2 = _conv_y(p2_ref, w2_ref, a1, c2, r)
    ym = y2 * mask
    sums_ref[0] = jnp.concatenate(
        [jnp.sum(ym, axis=1, keepdims=True),
         jnp.sum(ym * ym, axis=1, keepdims=True)], axis=1)


_PARALLEL = pltpu.CompilerParams(dimension_semantics=("parallel",))


def _unit_stats(x, wk, *, co, r, hp, wp, p):
    nb, c, l = x.shape
    return pl.pallas_call(
        functools.partial(_stats_body, c=c, r=r, hp=hp, wp=wp, p=p),
        grid=(nb,),
        in_specs=[pl.BlockSpec((1, c, l), lambda g: (g, 0, 0)),
                  pl.BlockSpec((co, 9 * c), lambda g: (0, 0))],
        out_specs=pl.BlockSpec((1, co, 2), lambda g: (g, 0, 0)),
        out_shape=jax.ShapeDtypeStruct((nb, co, 2), jnp.float32),
        scratch_shapes=[pltpu.VMEM((9 * c, l), jnp.float32)],
        compiler_params=_PARALLEL,
    )(x, wk)


def _unit_apply(x, wk, scale, shift, *, co, r, hp, wp, p):
    nb, c, l = x.shape
    return pl.pallas_call(
        functools.partial(_apply_body, c=c, r=r, hp=hp, wp=wp, p=p),
        grid=(nb,),
        in_specs=[pl.BlockSpec((1, c, l), lambda g: (g, 0, 0)),
                  pl.BlockSpec((co, 9 * c), lambda g: (0, 0)),
                  pl.BlockSpec((co, 1), lambda g: (0, 0)),
                  pl.BlockSpec((co, 1), lambda g: (0, 0))],
        out_specs=pl.BlockSpec((1, co, (hp - 2) * r), lambda g: (g, 0, 0)),
        out_shape=jax.ShapeDtypeStruct((nb, co, (hp - 2) * r), jnp.float32),
        scratch_shapes=[pltpu.VMEM((9 * c, l), jnp.float32)],
        compiler_params=_PARALLEL,
    )(x, wk, scale, shift)


def _unit_apply_stats(x, w1k, scale, shift, w2k, *, c1o, c2o, r, hp, wp, p):
    nb, c1, l = x.shape
    return pl.pallas_call(
        functools.partial(_apply_stats_body, c1=c1, c2=c1o, r=r, hp=hp,
                          wp=wp, p=p),
        grid=(nb,),
        in_specs=[pl.BlockSpec((1, c1, l), lambda g: (g, 0, 0)),
                  pl.BlockSpec((c1o, 9 * c1), lambda g: (0, 0)),
                  pl.BlockSpec((c1o, 1), lambda g: (0, 0)),
                  pl.BlockSpec((c1o, 1), lambda g: (0, 0)),
                  pl.BlockSpec((c2o, 9 * c1o), lambda g: (0, 0))],
        out_specs=(pl.BlockSpec((1, c1o, l), lambda g: (g, 0, 0)),
                   pl.BlockSpec((1, c2o, 2), lambda g: (g, 0, 0))),
        out_shape=(jax.ShapeDtypeStruct((nb, c1o, l), jnp.float32),
                   jax.ShapeDtypeStruct((nb, c2o, 2), jnp.float32)),
        scratch_shapes=[pltpu.VMEM((9 * c1, l), jnp.float32),
                        pltpu.VMEM((9 * c1o, l), jnp.float32)],
        compiler_params=_PARALLEL,
    )(x, w1k, scale, shift, w2k)


def _sel_matrix(r, r2, pool, slots):
    ki = lax.broadcasted_iota(jnp.int32, (r, r2), 0)
    ji = lax.broadcasted_iota(jnp.int32, (r, r2), 1)
    hit = None
    for in_base, out_base, w2 in slots:
        e = ((ji >= out_base) & (ji < out_base + w2)
             & (ki == (ji - out_base) * pool + in_base))
        hit = e if hit is None else (hit | e)
    return hit.astype(jnp.float32)


def _pool_pack_body(x_ref, o_ref, *, gin, c, h, r, pool, wpin, pin,
                    r2, wp2):
    h2 = h // pool
    w2 = (wpin - 2) // pool
    acc = None
    for q in range(gin):
        t = x_ref[q].reshape(c, h2, pool, r).max(axis=2)
        m = t
        for s in range(1, pool):
            m = jnp.maximum(m, pltpu.roll(t, r - s, 2))
        m2 = m.reshape(c * h2, r)
        slots = tuple((p * wpin + 1, (q * pin + p) * wp2 + 1, w2)
                      for p in range(pin))
        part = jnp.dot(m2, _sel_matrix(r, r2, pool, slots),
                       preferred_element_type=jnp.float32)
        acc = part if acc is None else acc + part
    mid = acc.reshape(c, h2, r2)
    z = jnp.zeros((c, 1, r2), jnp.float32)
    o_ref[0] = jnp.concatenate([z, mid, z], axis=1)


def _pool_pack(y, *, gin, pool, wpin, pin, r2, wp2):
    nbin, c, hr = y.shape
    r = 256 if wpin == 226 else 128
    h = hr // r
    hp2 = h // pool + 2
    x4 = y.reshape(nbin, c, h, r)
    out = pl.pallas_call(
        functools.partial(_pool_pack_body, gin=gin, c=c, h=h, r=r, pool=pool,
                          wpin=wpin, pin=pin, r2=r2, wp2=wp2),
        grid=(nbin // gin,),
        in_specs=[pl.BlockSpec((gin, c, h, r), lambda g: (g, 0, 0, 0))],
        out_specs=pl.BlockSpec((1, c, hp2, r2), lambda g: (g, 0, 0, 0)),
        out_shape=jax.ShapeDtypeStruct((nbin // gin, c, hp2, r2),
                                       jnp.float32),
        compiler_params=_PARALLEL,
    )(x4)
    return out.reshape(nbin // gin, c, hp2 * r2)


def _pool_last_body(x_ref, o_ref, *, c, h, r, pool, wpin, pin):
    h2 = h // pool
    w2 = (wpin - 2) // pool
    t = x_ref[0].reshape(c, h2, pool, r).max(axis=2)
    m = t
    for s in range(1, pool):
        m = jnp.maximum(m, pltpu.roll(t, r - s, 2))
    m2 = m.reshape(c * h2, r)
    for p in range(pin):
        sel = _sel_matrix(r, w2, pool, ((p * wpin + 1, 0, w2),))
        img = jnp.dot(m2, sel, preferred_element_type=jnp.float32)
        o_ref[p] = img.reshape(c, h2, w2)


def _pool_last(y, *, pool, wpin, pin):
    nbin, c, hr = y.shape
    r = 128
    h = hr // r
    h2, w2 = h // pool, (wpin - 2) // pool
    x4 = y.reshape(nbin, c, h, r)
    return pl.pallas_call(
        functools.partial(_pool_last_body, c=c, h=h, r=r, pool=pool,
                          wpin=wpin, pin=pin),
        grid=(nbin,),
        in_specs=[pl.BlockSpec((1, c, h, r), lambda g: (g, 0, 0, 0))],
        out_specs=pl.BlockSpec((pin, c, h2, w2), lambda g: (g, 0, 0, 0)),
        out_shape=jax.ShapeDtypeStruct((nbin * pin, c, h2, w2), jnp.float32),
        compiler_params=_PARALLEL,
    )(x4)


def _unit6_body(p_ref, w_ref, g_ref, b_ref, o_ref):
    y = jnp.dot(w_ref[...], p_ref[...], preferred_element_type=jnp.float32)
    m = y.shape[1]
    mean = jnp.sum(y, axis=1, keepdims=True) / m
    var = jnp.maximum(jnp.sum(y * y, axis=1, keepdims=True) / m - mean * mean,
                      0.0)
    scale = g_ref[...] * lax.rsqrt(var + _EPS)
    shift = b_ref[...] - mean * scale
    o_ref[...] = jnp.maximum(y * scale + shift, 0.0)


def _fc_body(x_ref, w_ref, b_ref, o_ref):
    o_ref[...] = jnp.dot(x_ref[...], w_ref[...],
                         preferred_element_type=jnp.float32) + b_ref[...]


def kernel(x, u1_w, u1_b, u1_g, u1_bt, u2_w, u2_b, u2_g, u2_bt,
           u3_w, u3_b, u3_g, u3_bt, u4_w, u4_b, u4_g, u4_bt,
           u5_w, u5_b, u5_g, u5_bt, u6_w, u6_b, u6_g, u6_bt,
           fc1_w, fc1_b, fc2_w, fc2_b):
    del u1_b, u2_b, u3_b, u4_b, u5_b, u6_b
    n = x.shape[0]
    w1k, w2k, w3k = _prep_w(u1_w), _prep_w(u2_w), _prep_w(u3_w)
    w4k, w5k = _prep_w(u4_w), _prep_w(u5_w)

    g1 = dict(r=256, hp=226, wp=226, p=1)
    m12 = float(n * 224 * 224)
    x1 = _pack(x, 256, 1)
    s1 = _unit_stats(x1, w1k, co=12, **g1)
    sc1, sh1 = _bn_coeffs(s1, u1_g, u1_bt, m12)
    a1, s2 = _unit_apply_stats(x1, w1k, sc1, sh1, w2k, c1o=12, c2o=24, **g1)
    sc2, sh2 = _bn_coeffs(s2, u2_g, u2_bt, m12)
    a2 = _unit_apply(a1, w2k, sc2, sh2, co=24, **g1)

    g3 = dict(r=128, hp=114, wp=114, p=1)
    x3 = _pool_pack(a2, gin=1, pool=2, wpin=226, pin=1, r2=128, wp2=114)
    s3 = _unit_stats(x3, w3k, co=36, **g3)
    sc3, sh3 = _bn_coeffs(s3, u3_g, u3_bt, float(n * 112 * 112))
    a3 = _unit_apply(x3, w3k, sc3, sh3, co=36, **g3)

    g4 = dict(r=128, hp=58, wp=58, p=2)
    x4 = _pool_pack(a3, gin=2, pool=2, wpin=114, pin=1, r2=128, wp2=58)
    s4 = _unit_stats(x4, w4k, co=48, **g4)
    sc4, sh4 = _bn_coeffs(s4, u4_g, u4_bt, float(n * 56 * 56))
    a4 = _unit_apply(x4, w4k, sc4, sh4, co=48, **g4)

    g5 = dict(r=128, hp=30, wp=30, p=4)
    x5 = _pool_pack(a4, gin=2, pool=2, wpin=58, pin=2, r2=128, wp2=30)
    s5 = _unit_stats(x5, w5k, co=48, **g5)
    sc5, sh5 = _bn_coeffs(s5, u5_g, u5_bt, float(n * 28 * 28))
    a5 = _unit_apply(x5, w5k, sc5, sh5, co=48, **g5)

    x6 = _pool_last(a5, pool=4, wpin=30, pin=4)
    taps = jnp.stack([x6[:, :, i:i + 3, j:j + 3]
                      for i in range(5) for j in range(5)], axis=0)
    p6 = jnp.transpose(taps, (0, 2, 1, 3, 4)).reshape(25 * 48, n * 9)
    w6k = jnp.transpose(u6_w, (0, 2, 3, 1)).reshape(96, 25 * 48)
    a6 = pl.pallas_call(
        _unit6_body,
        out_shape=jax.ShapeDtypeStruct((96, n * 9), jnp.float32),
    )(p6, w6k, u6_g.reshape(96, 1), u6_bt.reshape(96, 1))

    feat = jnp.transpose(a6.reshape(96, n, 9), (1, 0, 2)).reshape(n, 864)
    wf = jnp.concatenate([fc1_w, fc2_w], axis=0).T
    bf = jnp.concatenate([fc1_b, fc2_b]).reshape(1, 10)
    out = pl.pallas_call(
        _fc_body,
        out_shape=jax.ShapeDtypeStruct((n, 10), jnp.float32),
    )(feat, wf, bf)
    return out[:, :5], out[:, 5:]


# speedup vs baseline: 24.1397x; 1.0259x over previous
import functools

import jax
import jax.numpy as jnp
from jax import lax
from jax.experimental import pallas as pl
from jax.experimental.pallas import tpu as pltpu

_EPS = 1e-5
_BF = jnp.bfloat16


def _prep_w(w):
    co = w.shape[0]
    return jnp.transpose(w, (0, 2, 3, 1)).reshape(co, -1).astype(_BF)


def _bn_coeffs(sums, gamma, beta, m):
    s = jnp.sum(sums, axis=0)
    mean = s[:, 0] / m
    var = jnp.maximum(s[:, 1] / m - mean * mean, 0.0)
    scale = gamma * lax.rsqrt(var + _EPS)
    shift = beta - mean * scale
    return scale.reshape(-1, 1), shift.reshape(-1, 1)


def _real_mask(l, r, hp, wp, p):
    lane = lax.broadcasted_iota(jnp.int32, (1, l), 1)
    row = lax.shift_right_logical(lane, r.bit_length() - 1)
    col = jnp.bitwise_and(lane, r - 1)
    ok_h = (row >= 1) & (row <= hp - 2)
    ok_w = (col >= 1) & (col <= wp - 2)
    for q in range(1, p):
        ok_w |= (col >= q * wp + 1) & (col <= q * wp + wp - 2)
    return (ok_h & ok_w).astype(jnp.float32)


def _fill_taps(p_ref, x2, c, r):
    l = x2.shape[1]
    bases = (pltpu.roll(x2, 1, 1), x2, pltpu.roll(x2, l - 1, 1))
    for i in range(3):
        for j in range(3):
            tap = pltpu.roll(bases[j], (-(i - 1) * r) % l, 1)
            t = i * 3 + j
            p_ref[t * c:(t + 1) * c, :] = tap.astype(_BF)


def _conv_y(p_ref, w_ref, x2, c, r):
    _fill_taps(p_ref, x2, c, r)
    return jnp.dot(w_ref[...], p_ref[...], preferred_element_type=jnp.float32)


def _sums(ym):
    return jnp.concatenate(
        [jnp.sum(ym, axis=1, keepdims=True),
         jnp.sum(ym * ym, axis=1, keepdims=True)], axis=1)


def _stats_body(x_ref, w_ref, sums_ref, p_ref, *, c, r, hp, wp, p):
    x2 = x_ref[0].astype(jnp.float32)
    y = _conv_y(p_ref, w_ref, x2, c, r)
    sums_ref[0] = _sums(y * _real_mask(y.shape[1], r, hp, wp, p))


def _apply_body(x_ref, w_ref, sc_ref, sh_ref, o_ref, p_ref, *, c, r, hp, wp, p):
    x2 = x_ref[0].astype(jnp.float32)
    y = _conv_y(p_ref, w_ref, x2, c, r)
    mask = _real_mask(y.shape[1], r, hp, wp, p)
    a = jnp.maximum(y * sc_ref[...] + sh_ref[...], 0.0) * mask
    o_ref[0] = a[:, r:(hp - 1) * r].astype(_BF)


def _chain_body(x_ref, w1_ref, sc_ref, sh_ref, w2_ref, o_ref, p1_ref, p2_ref,
                *, c1, r, hp, wp, p, mode, sc2_ref=None, sh2_ref=None):
    x2 = x_ref[0].astype(jnp.float32)
    y1 = _conv_y(p1_ref, w1_ref, x2, c1, r)
    mask = _real_mask(y1.shape[1], r, hp, wp, p)
    a1 = jnp.maximum(y1 * sc_ref[...] + sh_ref[...], 0.0) * mask
    c2 = w1_ref.shape[0]
    y2 = _conv_y(p2_ref, w2_ref, a1, c2, r)
    if mode == "stats":
        o_ref[0] = _sums(y2 * mask)
    else:
        a2 = jnp.maximum(y2 * sc2_ref[...] + sh2_ref[...], 0.0) * mask
        o_ref[0] = a2[:, r:(hp - 1) * r].astype(_BF)


_PARALLEL = pltpu.CompilerParams(dimension_semantics=("parallel",))


def _unit_stats(x, wk, *, co, r, hp, wp, p):
    nb, c, l = x.shape
    return pl.pallas_call(
        functools.partial(_stats_body, c=c, r=r, hp=hp, wp=wp, p=p),
        grid=(nb,),
        in_specs=[pl.BlockSpec((1, c, l), lambda g: (g, 0, 0)),
                  pl.BlockSpec((co, 9 * c), lambda g: (0, 0))],
        out_specs=pl.BlockSpec((1, co, 2), lambda g: (g, 0, 0)),
        out_shape=jax.ShapeDtypeStruct((nb, co, 2), jnp.float32),
        scratch_shapes=[pltpu.VMEM((9 * c, l), _BF)],
        compiler_params=_PARALLEL,
    )(x, wk)


def _unit_apply(x, wk, scale, shift, *, co, r, hp, wp, p):
    nb, c, l = x.shape
    return pl.pallas_call(
        functools.partial(_apply_body, c=c, r=r, hp=hp, wp=wp, p=p),
        grid=(nb,),
        in_specs=[pl.BlockSpec((1, c, l), lambda g: (g, 0, 0)),
                  pl.BlockSpec((co, 9 * c), lambda g: (0, 0)),
                  pl.BlockSpec((co, 1), lambda g: (0, 0)),
                  pl.BlockSpec((co, 1), lambda g: (0, 0))],
        out_specs=pl.BlockSpec((1, co, (hp - 2) * r), lambda g: (g, 0, 0)),
        out_shape=jax.ShapeDtypeStruct((nb, co, (hp - 2) * r), _BF),
        scratch_shapes=[pltpu.VMEM((9 * c, l), _BF)],
        compiler_params=_PARALLEL,
    )(x, wk, scale, shift)


def _unit12_chain(x, w1k, sc1, sh1, w2k, *, c1o, c2o, r, hp, wp, p,
                  sc2=None, sh2=None):
    nb, c1, l = x.shape
    mode = "stats" if sc2 is None else "apply"
    if mode == "stats":
        out_spec = pl.BlockSpec((1, c2o, 2), lambda g: (g, 0, 0))
        out_shape = jax.ShapeDtypeStruct((nb, c2o, 2), jnp.float32)
        extra = []
    else:
        out_spec = pl.BlockSpec((1, c2o, (hp - 2) * r), lambda g: (g, 0, 0))
        out_shape = jax.ShapeDtypeStruct((nb, c2o, (hp - 2) * r), _BF)
        extra = [sc2, sh2]

    def body(x_ref, w1_ref, sc_ref, sh_ref, w2_ref, *rest):
        if mode == "stats":
            o_ref, p1_ref, p2_ref = rest
            _chain_body(x_ref, w1_ref, sc_ref, sh_ref, w2_ref, o_ref,
                        p1_ref, p2_ref, c1=c1, r=r, hp=hp, wp=wp, p=p,
                        mode=mode)
        else:
            sc2_ref, sh2_ref, o_ref, p1_ref, p2_ref = rest
            _chain_body(x_ref, w1_ref, sc_ref, sh_ref, w2_ref, o_ref,
                        p1_ref, p2_ref, c1=c1, r=r, hp=hp, wp=wp, p=p,
                        mode=mode, sc2_ref=sc2_ref, sh2_ref=sh2_ref)

    return pl.pallas_call(
        body,
        grid=(nb,),
        in_specs=[pl.BlockSpec((1, c1, l), lambda g: (g, 0, 0)),
                  pl.BlockSpec((c1o, 9 * c1), lambda g: (0, 0)),
                  pl.BlockSpec((c1o, 1), lambda g: (0, 0)),
                  pl.BlockSpec((c1o, 1), lambda g: (0, 0)),
                  pl.BlockSpec((c2o, 9 * c1o), lambda g: (0, 0))]
                 + [pl.BlockSpec((c2o, 1), lambda g: (0, 0))] * len(extra),
        out_specs=out_spec,
        out_shape=out_shape,
        scratch_shapes=[pltpu.VMEM((9 * c1, l), _BF),
                        pltpu.VMEM((9 * c1o, l), _BF)],
        compiler_params=_PARALLEL,
    )(x, w1k, sc1, sh1, w2k, *extra)


def _sel_matrix(r, r2, pool, slots):
    ki = lax.broadcasted_iota(jnp.int32, (r, r2), 0)
    ji = lax.broadcasted_iota(jnp.int32, (r, r2), 1)
    hit = None
    for in_base, out_base, w2 in slots:
        e = ((ji >= out_base) & (ji < out_base + w2)
             & (ki == (ji - out_base) * pool + in_base))
        hit = e if hit is None else (hit | e)
    return hit.astype(jnp.float32)


def _pooled_rows(x_q, c, h, r, pool):
    h2 = h // pool
    t = x_q.reshape(c, h2, pool, r).max(axis=2)
    m = t
    for s in range(1, pool):
        m = jnp.maximum(m, pltpu.roll(t, r - s, 2))
    return m.reshape(c * h2, r)


def _pool_pack_body(x_ref, o_ref, *, gin, c, h, r, pool, wpin, pin, r2, wp2):
    h2 = h // pool
    w2 = (wpin - 2) // pool
    acc = None
    for q in range(gin):
        m2 = _pooled_rows(x_ref[q].astype(jnp.float32), c, h, r, pool)
        slots = tuple((p * wpin + 1, (q * pin + p) * wp2 + 1, w2)
                      for p in range(pin))
        part = jnp.dot(m2, _sel_matrix(r, r2, pool, slots),
                       preferred_element_type=jnp.float32)
        acc = part if acc is None else acc + part
    mid = acc.reshape(c, h2, r2)
    z = jnp.zeros((c, 1, r2), jnp.float32)
    o_ref[0] = jnp.concatenate([z, mid, z], axis=1).astype(_BF)


def _pool_pack(y, *, gin, pool, wpin, pin, r2, wp2):
    nbin, c, hr = y.shape
    r = 256 if wpin == 226 else 128
    h = hr // r
    hp2 = h // pool + 2
    x4 = y.reshape(nbin, c, h, r)
    out = pl.pallas_call(
        functools.partial(_pool_pack_body, gin=gin, c=c, h=h, r=r, pool=pool,
                          wpin=wpin, pin=pin, r2=r2, wp2=wp2),
        grid=(nbin // gin,),
        in_specs=[pl.BlockSpec((gin, c, h, r), lambda g: (g, 0, 0, 0))],
        out_specs=pl.BlockSpec((1, c, hp2, r2), lambda g: (g, 0, 0, 0)),
        out_shape=jax.ShapeDtypeStruct((nbin // gin, c, hp2, r2), _BF),
        compiler_params=_PARALLEL,
    )(x4)
    return out.reshape(nbin // gin, c, hp2 * r2)


def _pool_last_body(x_ref, o_ref, *, c, h, r, pool, wpin, pin):
    h2 = h // pool
    w2 = (wpin - 2) // pool
    m2 = _pooled_rows(x_ref[0].astype(jnp.float32), c, h, r, pool)
    for p in range(pin):
        sel = _sel_matrix(r, w2, pool, ((p * wpin + 1, 0, w2),))
        img = jnp.dot(m2, sel, preferred_element_type=jnp.float32)
        o_ref[p] = img.reshape(c, h2, w2).astype(_BF)


def _pool_last(y, *, pool, wpin, pin):
    nbin, c, hr = y.shape
    r = 128
    h = hr // r
    h2, w2 = h // pool, (wpin - 2) // pool
    x4 = y.reshape(nbin, c, h, r)
    return pl.pallas_call(
        functools.partial(_pool_last_body, c=c, h=h, r=r, pool=pool,
                          wpin=wpin, pin=pin),
        grid=(nbin,),
        in_specs=[pl.BlockSpec((1, c, h, r), lambda g: (g, 0, 0, 0))],
        out_specs=pl.BlockSpec((pin, c, h2, w2), lambda g: (g, 0, 0, 0)),
        out_shape=jax.ShapeDtypeStruct((nbin * pin, c, h2, w2), _BF),
        compiler_params=_PARALLEL,
    )(x4)


def _pack_body(x_ref, o_ref):
    o_ref[...] = jnp.zeros_like(o_ref)
    o_ref[0, :, 1:225, 1:225] = x_ref[0].astype(_BF)


def _pack_init(x):
    n = x.shape[0]
    out = pl.pallas_call(
        _pack_body,
        grid=(n,),
        in_specs=[pl.BlockSpec((1, 3, 224, 224), lambda g: (g, 0, 0, 0))],
        out_specs=pl.BlockSpec((1, 3, 226, 256), lambda g: (g, 0, 0, 0)),
        out_shape=jax.ShapeDtypeStruct((n, 3, 226, 256), _BF),
        compiler_params=_PARALLEL,
    )(x)
    return out.reshape(n, 3, 226 * 256)


def _unit6_body(p_ref, w_ref, g_ref, b_ref, o_ref):
    y = jnp.dot(w_ref[...], p_ref[...], preferred_element_type=jnp.float32)
    m = y.shape[1]
    mean = jnp.sum(y, axis=1, keepdims=True) / m
    var = jnp.maximum(jnp.sum(y * y, axis=1, keepdims=True) / m - mean * mean,
                      0.0)
    scale = g_ref[...] * lax.rsqrt(var + _EPS)
    shift = b_ref[...] - mean * scale
    o_ref[...] = jnp.maximum(y * scale + shift, 0.0).astype(_BF)


def _fc_body(x_ref, w_ref, b_ref, o_ref):
    o_ref[...] = jnp.dot(x_ref[...], w_ref[...],
                         preferred_element_type=jnp.float32) + b_ref[...]


def kernel(x, u1_w, u1_b, u1_g, u1_bt, u2_w, u2_b, u2_g, u2_bt,
           u3_w, u3_b, u3_g, u3_bt, u4_w, u4_b, u4_g, u4_bt,
           u5_w, u5_b, u5_g, u5_bt, u6_w, u6_b, u6_g, u6_bt,
           fc1_w, fc1_b, fc2_w, fc2_b):
    del u1_b, u2_b, u3_b, u4_b, u5_b, u6_b
    n = x.shape[0]
    w1k, w2k, w3k = _prep_w(u1_w), _prep_w(u2_w), _prep_w(u3_w)
    w4k, w5k = _prep_w(u4_w), _prep_w(u5_w)

    g1 = dict(r=256, hp=226, wp=226, p=1)
    m12 = float(n * 224 * 224)
    x1 = _pack_init(x)
    s1 = _unit_stats(x1, w1k, co=12, **g1)
    sc1, sh1 = _bn_coeffs(s1, u1_g, u1_bt, m12)
    s2 = _unit12_chain(x1, w1k, sc1, sh1, w2k, c1o=12, c2o=24, **g1)
    sc2, sh2 = _bn_coeffs(s2, u2_g, u2_bt, m12)
    a2 = _unit12_chain(x1, w1k, sc1, sh1, w2k, c1o=12, c2o=24, **g1,
                       sc2=sc2, sh2=sh2)

    g3 = dict(r=128, hp=114, wp=114, p=1)
    x3 = _pool_pack(a2, gin=1, pool=2, wpin=226, pin=1, r2=128, wp2=114)
    s3 = _unit_stats(x3, w3k, co=36, **g3)
    sc3, sh3 = _bn_coeffs(s3, u3_g, u3_bt, float(n * 112 * 112))
    a3 = _unit_apply(x3, w3k, sc3, sh3, co=36, **g3)

    g4 = dict(r=128, hp=58, wp=58, p=2)
    x4 = _pool_pack(a3, gin=2, pool=2, wpin=114, pin=1, r2=128, wp2=58)
    s4 = _unit_stats(x4, w4k, co=48, **g4)
    sc4, sh4 = _bn_coeffs(s4, u4_g, u4_bt, float(n * 56 * 56))
    a4 = _unit_apply(x4, w4k, sc4, sh4, co=48, **g4)

    g5 = dict(r=128, hp=30, wp=30, p=4)
    x5 = _pool_pack(a4, gin=2, pool=2, wpin=58, pin=2, r2=128, wp2=30)
    s5 = _unit_stats(x5, w5k, co=48, **g5)
    sc5, sh5 = _bn_coeffs(s5, u5_g, u5_bt, float(n * 28 * 28))
    a5 = _unit_apply(x5, w5k, sc5, sh5, co=48, **g5)

    x6 = _pool_last(a5, pool=4, wpin=30, pin=4)
    taps = jnp.stack([x6[:, :, i:i + 3, j:j + 3]
                      for i in range(5) for j in range(5)], axis=0)
    p6 = jnp.transpose(taps, (0, 2, 1, 3, 4)).reshape(25 * 48, n * 9)
    w6k = jnp.transpose(u6_w, (0, 2, 3, 1)).reshape(96, 25 * 48).astype(_BF)
    a6 = pl.pallas_call(
        _unit6_body,
        out_shape=jax.ShapeDtypeStruct((96, n * 9), _BF),
    )(p6, w6k, u6_g.reshape(96, 1), u6_bt.reshape(96, 1))

    feat = jnp.transpose(a6.reshape(96, n, 9), (1, 0, 2)).reshape(n, 864)
    wf = jnp.concatenate([fc1_w, fc2_w], axis=0).T.astype(_BF)
    bf = jnp.concatenate([fc1_b, fc2_b]).reshape(1, 10)
    out = pl.pallas_call(
        _fc_body,
        out_shape=jax.ShapeDtypeStruct((n, 10), jnp.float32),
    )(feat, wf, bf)
    return out[:, :5], out[:, 5:]


# speedup vs baseline: 25.4082x; 1.0525x over previous
import functools

import jax
import jax.numpy as jnp
from jax import lax
from jax.experimental import pallas as pl
from jax.experimental.pallas import tpu as pltpu

_EPS = 1e-5
_BF = jnp.bfloat16


def _prep_w(w):
    co = w.shape[0]
    return jnp.transpose(w, (0, 2, 3, 1)).reshape(co, -1).astype(_BF)


def _bn_coeffs(sums, gamma, beta, m):
    s = jnp.sum(sums, axis=0)
    mean = s[:, 0] / m
    var = jnp.maximum(s[:, 1] / m - mean * mean, 0.0)
    scale = gamma * lax.rsqrt(var + _EPS)
    shift = beta - mean * scale
    return scale.reshape(-1, 1), shift.reshape(-1, 1)


def _real_mask(l, r, hp, wp, p):
    lane = lax.broadcasted_iota(jnp.int32, (1, l), 1)
    row = lax.shift_right_logical(lane, r.bit_length() - 1)
    col = jnp.bitwise_and(lane, r - 1)
    ok_h = (row >= 1) & (row <= hp - 2)
    ok_w = (col >= 1) & (col <= wp - 2)
    for q in range(1, p):
        ok_w |= (col >= q * wp + 1) & (col <= q * wp + wp - 2)
    return (ok_h & ok_w).astype(jnp.float32)


def _fill_taps(p_ref, x2, c, r):
    l = x2.shape[1]
    bases = (pltpu.roll(x2, 1, 1), x2, pltpu.roll(x2, l - 1, 1))
    for i in range(3):
        for j in range(3):
            tap = pltpu.roll(bases[j], (-(i - 1) * r) % l, 1)
            t = i * 3 + j
            p_ref[t * c:(t + 1) * c, :] = tap.astype(_BF)


def _conv_y(p_ref, w_ref, x2, c, r):
    _fill_taps(p_ref, x2, c, r)
    return jnp.dot(w_ref[...], p_ref[...], preferred_element_type=jnp.float32)


def _sums(ym):
    return jnp.concatenate(
        [jnp.sum(ym, axis=1, keepdims=True),
         jnp.sum(ym * ym, axis=1, keepdims=True)], axis=1)


def _stats_body(x_ref, w_ref, sums_ref, p_ref, *, c, r, hp, wp, p):
    x2 = x_ref[0].astype(jnp.float32)
    y = _conv_y(p_ref, w_ref, x2, c, r)
    sums_ref[0] = _sums(y * _real_mask(y.shape[1], r, hp, wp, p))


def _apply_body(x_ref, w_ref, sc_ref, sh_ref, o_ref, p_ref, *, c, r, hp, wp, p):
    x2 = x_ref[0].astype(jnp.float32)
    y = _conv_y(p_ref, w_ref, x2, c, r)
    mask = _real_mask(y.shape[1], r, hp, wp, p)
    a = jnp.maximum(y * sc_ref[...] + sh_ref[...], 0.0) * mask
    o_ref[0] = a[:, r:(hp - 1) * r].astype(_BF)


def _chain_body(x_ref, w1_ref, sc_ref, sh_ref, w2_ref, o_ref, p1_ref, p2_ref,
                *, c1, r, hp, wp, p, mode, sc2_ref=None, sh2_ref=None):
    x2 = x_ref[0].astype(jnp.float32)
    y1 = _conv_y(p1_ref, w1_ref, x2, c1, r)
    mask = _real_mask(y1.shape[1], r, hp, wp, p)
    a1 = jnp.maximum(y1 * sc_ref[...] + sh_ref[...], 0.0) * mask
    c2 = w1_ref.shape[0]
    y2 = _conv_y(p2_ref, w2_ref, a1, c2, r)
    if mode == "stats":
        o_ref[0] = _sums(y2 * mask)
    else:
        a2 = jnp.maximum(y2 * sc2_ref[...] + sh2_ref[...], 0.0) * mask
        o_ref[0] = a2[:, r:(hp - 1) * r].astype(_BF)


_PARALLEL = pltpu.CompilerParams(dimension_semantics=("parallel",))


def _unit_stats(x, wk, *, co, r, hp, wp, p):
    nb, c, l = x.shape
    return pl.pallas_call(
        functools.partial(_stats_body, c=c, r=r, hp=hp, wp=wp, p=p),
        grid=(nb,),
        in_specs=[pl.BlockSpec((1, c, l), lambda g: (g, 0, 0)),
                  pl.BlockSpec((co, 9 * c), lambda g: (0, 0))],
        out_specs=pl.BlockSpec((1, co, 2), lambda g: (g, 0, 0)),
        out_shape=jax.ShapeDtypeStruct((nb, co, 2), jnp.float32),
        scratch_shapes=[pltpu.VMEM((9 * c, l), _BF)],
        compiler_params=_PARALLEL,
    )(x, wk)


def _unit_apply(x, wk, scale, shift, *, co, r, hp, wp, p):
    nb, c, l = x.shape
    return pl.pallas_call(
        functools.partial(_apply_body, c=c, r=r, hp=hp, wp=wp, p=p),
        grid=(nb,),
        in_specs=[pl.BlockSpec((1, c, l), lambda g: (g, 0, 0)),
                  pl.BlockSpec((co, 9 * c), lambda g: (0, 0)),
                  pl.BlockSpec((co, 1), lambda g: (0, 0)),
                  pl.BlockSpec((co, 1), lambda g: (0, 0))],
        out_specs=pl.BlockSpec((1, co, (hp - 2) * r), lambda g: (g, 0, 0)),
        out_shape=jax.ShapeDtypeStruct((nb, co, (hp - 2) * r), _BF),
        scratch_shapes=[pltpu.VMEM((9 * c, l), _BF)],
        compiler_params=_PARALLEL,
    )(x, wk, scale, shift)


def _unit12_chain(x, w1k, sc1, sh1, w2k, *, c1o, c2o, r, hp, wp, p,
                  sc2=None, sh2=None):
    nb, c1, l = x.shape
    mode = "stats" if sc2 is None else "apply"
    if mode == "stats":
        out_spec = pl.BlockSpec((1, c2o, 2), lambda g: (g, 0, 0))
        out_shape = jax.ShapeDtypeStruct((nb, c2o, 2), jnp.float32)
        extra = []
    else:
        out_spec = pl.BlockSpec((1, c2o, (hp - 2) * r), lambda g: (g, 0, 0))
        out_shape = jax.ShapeDtypeStruct((nb, c2o, (hp - 2) * r), _BF)
        extra = [sc2, sh2]

    def body(x_ref, w1_ref, sc_ref, sh_ref, w2_ref, *rest):
        if mode == "stats":
            o_ref, p1_ref, p2_ref = rest
            _chain_body(x_ref, w1_ref, sc_ref, sh_ref, w2_ref, o_ref,
                        p1_ref, p2_ref, c1=c1, r=r, hp=hp, wp=wp, p=p,
                        mode=mode)
        else:
            sc2_ref, sh2_ref, o_ref, p1_ref, p2_ref = rest
            _chain_body(x_ref, w1_ref, sc_ref, sh_ref, w2_ref, o_ref,
                        p1_ref, p2_ref, c1=c1, r=r, hp=hp, wp=wp, p=p,
                        mode=mode, sc2_ref=sc2_ref, sh2_ref=sh2_ref)

    return pl.pallas_call(
        body,
        grid=(nb,),
        in_specs=[pl.BlockSpec((1, c1, l), lambda g: (g, 0, 0)),
                  pl.BlockSpec((c1o, 9 * c1), lambda g: (0, 0)),
                  pl.BlockSpec((c1o, 1), lambda g: (0, 0)),
                  pl.BlockSpec((c1o, 1), lambda g: (0, 0)),
                  pl.BlockSpec((c2o, 9 * c1o), lambda g: (0, 0))]
                 + [pl.BlockSpec((c2o, 1), lambda g: (0, 0))] * len(extra),
        out_specs=out_spec,
        out_shape=out_shape,
        scratch_shapes=[pltpu.VMEM((9 * c1, l), _BF),
                        pltpu.VMEM((9 * c1o, l), _BF)],
        compiler_params=_PARALLEL,
    )(x, w1k, sc1, sh1, w2k, *extra)


def _sel_matrix(r, r2, pool, slots):
    ki = lax.broadcasted_iota(jnp.int32, (r, r2), 0)
    ji = lax.broadcasted_iota(jnp.int32, (r, r2), 1)
    hit = None
    for in_base, out_base, w2 in slots:
        e = ((ji >= out_base) & (ji < out_base + w2)
             & (ki == (ji - out_base) * pool + in_base))
        hit = e if hit is None else (hit | e)
    return hit.astype(jnp.float32)


def _pooled_rows(x_q, c, h, r, pool):
    h2 = h // pool
    t = x_q.reshape(c, h2, pool, r).max(axis=2)
    m = t
    for s in range(1, pool):
        m = jnp.maximum(m, pltpu.roll(t, r - s, 2))
    return m.reshape(c * h2, r)


def _pool_pack_body(x_ref, o_ref, *, gin, c, h, r, pool, wpin, pin, r2, wp2):
    h2 = h // pool
    w2 = (wpin - 2) // pool
    acc = None
    for q in range(gin):
        m2 = _pooled_rows(x_ref[q].astype(jnp.float32), c, h, r, pool)
        slots = tuple((p * wpin + 1, (q * pin + p) * wp2 + 1, w2)
                      for p in range(pin))
        part = jnp.dot(m2, _sel_matrix(r, r2, pool, slots),
                       preferred_element_type=jnp.float32)
        acc = part if acc is None else acc + part
    mid = acc.reshape(c, h2, r2)
    hp2s = -(-(h2 + 2) // 8) * 8
    z = jnp.zeros((c, 1, r2), jnp.float32)
    zb = jnp.zeros((c, hp2s - 1 - h2, r2), jnp.float32)
    o_ref[0] = jnp.concatenate([z, mid, zb], axis=1).astype(_BF)


def _pool_pack(y, *, gin, pool, wpin, pin, r2, wp2):
    nbin, c, hr = y.shape
    r = 256 if wpin == 226 else 128
    h = hr // r
    hp2s = -(-(h // pool + 2) // 8) * 8
    x4 = y.reshape(nbin, c, h, r)
    out = pl.pallas_call(
        functools.partial(_pool_pack_body, gin=gin, c=c, h=h, r=r, pool=pool,
                          wpin=wpin, pin=pin, r2=r2, wp2=wp2),
        grid=(nbin // gin,),
        in_specs=[pl.BlockSpec((gin, c, h, r), lambda g: (g, 0, 0, 0))],
        out_specs=pl.BlockSpec((1, c, hp2s, r2), lambda g: (g, 0, 0, 0)),
        out_shape=jax.ShapeDtypeStruct((nbin // gin, c, hp2s, r2), _BF),
        compiler_params=_PARALLEL,
    )(x4)
    return out.reshape(nbin // gin, c, hp2s * r2)


def _pool_last_body(x_ref, w6_ref, o_ref, *, c, h, r, pool, wpin, pin, ks):
    h2 = h // pool
    w2 = (wpin - 2) // pool
    ho = h2 - ks + 1
    m2 = _pooled_rows(x_ref[0].astype(jnp.float32), c, h, r, pool)
    for p in range(pin):
        sel = _sel_matrix(r, w2, pool, ((p * wpin + 1, 0, w2),))
        img = jnp.dot(m2, sel, preferred_element_type=jnp.float32)
        img = img.reshape(c, h2, w2)
        pats = []
        for i in range(ks):
            for j in range(ks):
                pats.append(jnp.concatenate(
                    [img[:, i + hh, j:j + ho] for hh in range(ho)], axis=1))
        pat = jnp.concatenate(pats, axis=0).astype(_BF)
        o_ref[p] = jnp.dot(w6_ref[...], pat,
                           preferred_element_type=jnp.float32)


def _pool_last(y, w6k, *, pool, wpin, pin, ks):
    nbin, c, hr = y.shape
    r = 128
    h = hr // r
    h2, w2 = h // pool, (wpin - 2) // pool
    ho = h2 - ks + 1
    co = w6k.shape[0]
    x4 = y.reshape(nbin, c, h, r)
    return pl.pallas_call(
        functools.partial(_pool_last_body, c=c, h=h, r=r, pool=pool,
                          wpin=wpin, pin=pin, ks=ks),
        grid=(nbin,),
        in_specs=[pl.BlockSpec((1, c, h, r), lambda g: (g, 0, 0, 0)),
                  pl.BlockSpec((co, ks * ks * c), lambda g: (0, 0))],
        out_specs=pl.BlockSpec((pin, co, ho * ho), lambda g: (g, 0, 0)),
        out_shape=jax.ShapeDtypeStruct((nbin * pin, co, ho * ho),
                                       jnp.float32),
        compiler_params=_PARALLEL,
    )(x4, w6k)


def _pack_body(x_ref, o_ref):
    o_ref[...] = jnp.zeros_like(o_ref)
    o_ref[0, :, 1:225, 1:225] = x_ref[0].astype(_BF)


def _pack_init(x):
    n = x.shape[0]
    out = pl.pallas_call(
        _pack_body,
        grid=(n,),
        in_specs=[pl.BlockSpec((1, 3, 224, 224), lambda g: (g, 0, 0, 0))],
        out_specs=pl.BlockSpec((1, 3, 232, 256), lambda g: (g, 0, 0, 0)),
        out_shape=jax.ShapeDtypeStruct((n, 3, 232, 256), _BF),
        compiler_params=_PARALLEL,
    )(x)
    return out.reshape(n, 3, 232 * 256)


def _unit6_bn_body(y_ref, g_ref, b_ref, o_ref):
    y = y_ref[...]
    m = y.shape[0] * y.shape[2]
    mean = jnp.sum(y, axis=(0, 2), keepdims=True) / m
    var = jnp.maximum(
        jnp.sum(y * y, axis=(0, 2), keepdims=True) / m - mean * mean, 0.0)
    scale = g_ref[...].reshape(1, -1, 1) * lax.rsqrt(var + _EPS)
    shift = b_ref[...].reshape(1, -1, 1) - mean * scale
    o_ref[...] = jnp.maximum(y * scale + shift, 0.0).astype(_BF)


def _fc_body(x_ref, w_ref, b_ref, o_ref):
    o_ref[...] = jnp.dot(x_ref[...], w_ref[...],
                         preferred_element_type=jnp.float32) + b_ref[...]


def kernel(x, u1_w, u1_b, u1_g, u1_bt, u2_w, u2_b, u2_g, u2_bt,
           u3_w, u3_b, u3_g, u3_bt, u4_w, u4_b, u4_g, u4_bt,
           u5_w, u5_b, u5_g, u5_bt, u6_w, u6_b, u6_g, u6_bt,
           fc1_w, fc1_b, fc2_w, fc2_b):
    del u1_b, u2_b, u3_b, u4_b, u5_b, u6_b
    n = x.shape[0]
    w1k, w2k, w3k = _prep_w(u1_w), _prep_w(u2_w), _prep_w(u3_w)
    w4k, w5k = _prep_w(u4_w), _prep_w(u5_w)

    g1 = dict(r=256, hp=226, wp=226, p=1)
    m12 = float(n * 224 * 224)
    x1 = _pack_init(x)
    s1 = _unit_stats(x1, w1k, co=12, **g1)
    sc1, sh1 = _bn_coeffs(s1, u1_g, u1_bt, m12)
    s2 = _unit12_chain(x1, w1k, sc1, sh1, w2k, c1o=12, c2o=24, **g1)
    sc2, sh2 = _bn_coeffs(s2, u2_g, u2_bt, m12)
    a2 = _unit12_chain(x1, w1k, sc1, sh1, w2k, c1o=12, c2o=24, **g1,
                       sc2=sc2, sh2=sh2)

    g3 = dict(r=128, hp=114, wp=114, p=1)
    x3 = _pool_pack(a2, gin=1, pool=2, wpin=226, pin=1, r2=128, wp2=114)
    s3 = _unit_stats(x3, w3k, co=36, **g3)
    sc3, sh3 = _bn_coeffs(s3, u3_g, u3_bt, float(n * 112 * 112))
    a3 = _unit_apply(x3, w3k, sc3, sh3, co=36, **g3)

    g4 = dict(r=128, hp=58, wp=58, p=2)
    x4 = _pool_pack(a3, gin=2, pool=2, wpin=114, pin=1, r2=128, wp2=58)
    s4 = _unit_stats(x4, w4k, co=48, **g4)
    sc4, sh4 = _bn_coeffs(s4, u4_g, u4_bt, float(n * 56 * 56))
    a4 = _unit_apply(x4, w4k, sc4, sh4, co=48, **g4)

    g5 = dict(r=128, hp=30, wp=30, p=4)
    x5 = _pool_pack(a4, gin=2, pool=2, wpin=58, pin=2, r2=128, wp2=30)
    s5 = _unit_stats(x5, w5k, co=48, **g5)
    sc5, sh5 = _bn_coeffs(s5, u5_g, u5_bt, float(n * 28 * 28))
    a5 = _unit_apply(x5, w5k, sc5, sh5, co=48, **g5)

    y6 = _pool_last(a5, _prep_w(u6_w), pool=4, wpin=30, pin=4, ks=5)
    a6 = pl.pallas_call(
        _unit6_bn_body,
        out_shape=jax.ShapeDtypeStruct((n, 96, 9), _BF),
    )(y6, u6_g.reshape(96, 1), u6_bt.reshape(96, 1))

    feat = a6.reshape(n, 864)
    wf = jnp.concatenate([fc1_w, fc2_w], axis=0).T.astype(_BF)
    bf = jnp.concatenate([fc1_b, fc2_b]).reshape(1, 10)
    out = pl.pallas_call(
        _fc_body,
        out_shape=jax.ShapeDtypeStruct((n, 10), jnp.float32),
    )(feat, wf, bf)
    return out[:, :5], out[:, 5:]
